# Initial kernel scaffold; baseline (speedup 1.0000x reference)
import jax
import jax.numpy as jnp
from jax.experimental import pallas as pl


def kernel(x, Wg, W1, b1, W2, b2):
    raise NotImplementedError("write your pallas kernel here")


# speedup vs baseline: 1.1170x; 1.1170x over previous
import functools

import jax
import jax.numpy as jnp
from jax import lax
from jax.experimental import pallas as pl
from jax.experimental.pallas import tpu as pltpu
from jax.experimental.pallas import tpu_sc as plsc

_NC, _NS = 2, 16
_NW = _NC * _NS

_BT = 256
_BTG = 512
_GC = 64
_CC = 32


def _gate_body(e_static, x_ref, wg_ref, e0_ref, e1_ref, w0_ref, w1_ref):
    logits = jnp.dot(x_ref[...], wg_ref[...], preferred_element_type=jnp.float32)
    col = lax.broadcasted_iota(jnp.int32, logits.shape, 1)
    neg = jnp.float32(-1e30)
    l0 = jnp.where(col < e_static, logits, neg)
    m1 = jnp.max(l0, axis=1, keepdims=True)
    i1 = jnp.min(jnp.where(l0 == m1, col, logits.shape[1]), axis=1, keepdims=True)
    lm = jnp.where(col == i1, neg, l0)
    m2 = jnp.max(lm, axis=1, keepdims=True)
    i2 = jnp.min(jnp.where(lm == m2, col, logits.shape[1]), axis=1, keepdims=True)
    t = jnp.exp(m2 - m1)
    w0 = 1.0 / (1.0 + t)
    e0_ref[0, 0, :] = i1[:, 0]
    e1_ref[0, 0, :] = i2[:, 0]
    w0_ref[0, 0, :] = w0[:, 0]
    w1_ref[0, 0, :] = 1.0 - w0[:, 0]


def _ffn_body(be_ref, xs_ref, w1_ref, b1_ref, w2_ref, b2_ref, ws_ref, ys_ref):
    xb = xs_ref[...]
    h = jnp.dot(xb, w1_ref[0], preferred_element_type=jnp.float32) + b1_ref[0]
    g = jax.nn.gelu(h)
    y = jnp.dot(g, w2_ref[0], preferred_element_type=jnp.float32) + b2_ref[0]
    ys_ref[...] = y * ws_ref[0, 0][:, None]


def _gating(xf, Wg):
    t, d = xf.shape
    e = Wg.shape[1]
    lanes = 128
    wg_pad = jnp.pad(Wg, ((0, 0), (0, lanes - e)))
    nbg = t // _BTG
    e0, e1, w0, w1 = pl.pallas_call(
        functools.partial(_gate_body, e),
        grid=(nbg,),
        in_specs=[
            pl.BlockSpec((_BTG, d), lambda i: (i, 0)),
            pl.BlockSpec((d, lanes), lambda i: (0, 0)),
        ],
        out_specs=[
            pl.BlockSpec((1, 1, _BTG), lambda i: (i, 0, 0)),
            pl.BlockSpec((1, 1, _BTG), lambda i: (i, 0, 0)),
            pl.BlockSpec((1, 1, _BTG), lambda i: (i, 0, 0)),
            pl.BlockSpec((1, 1, _BTG), lambda i: (i, 0, 0)),
        ],
        out_shape=[
            jax.ShapeDtypeStruct((nbg, 1, _BTG), jnp.int32),
            jax.ShapeDtypeStruct((nbg, 1, _BTG), jnp.int32),
            jax.ShapeDtypeStruct((nbg, 1, _BTG), jnp.float32),
            jax.ShapeDtypeStruct((nbg, 1, _BTG), jnp.float32),
        ],
    )(xf, wg_pad)
    return e0.reshape(t), e1.reshape(t), w0.reshape(t), w1.reshape(t)


def _route(e0, e1, w0, w1, e):
    t = e0.shape[0]
    p = t * 2
    nb = p // _BT + e
    npos = nb * _BT
    flat_e = jnp.stack([e0, e1], axis=1).reshape(p)
    flat_w = jnp.stack([w0, w1], axis=1).reshape(p)
    oh = (flat_e[:, None] == jnp.arange(e, dtype=jnp.int32)[None, :]).astype(jnp.int32)
    csum = jnp.cumsum(oh, axis=0)
    counts = csum[-1]
    rank = jnp.take_along_axis(csum, flat_e[:, None], axis=1)[:, 0] - 1
    nblk = (counts + _BT - 1) // _BT
    blk_start = jnp.concatenate(
        [jnp.zeros((1,), jnp.int32), jnp.cumsum(nblk)[:-1].astype(jnp.int32)])
    pos = blk_start[flat_e] * _BT + rank
    tok = jnp.arange(p, dtype=jnp.int32) // 2
    tok_pos = jnp.zeros((npos,), jnp.int32).at[pos].set(tok)
    ws_pos = jnp.zeros((npos,), jnp.float32).at[pos].set(flat_w)
    block_expert = jnp.minimum(
        (jnp.arange(nb, dtype=jnp.int32)[:, None] >= blk_start[None, :])
        .astype(jnp.int32).sum(axis=1) - 1,
        e - 1).astype(jnp.int32)
    pos2 = pos.reshape(t, 2)
    return tok_pos, ws_pos, block_expert, pos2[:, 0], pos2[:, 1]


def _ffn(block_expert, xs, W1, b1, W2, b2, ws_pos):
    npos, d = xs.shape
    e, _, f = W1.shape
    nb = npos // _BT
    ws3 = ws_pos.reshape(nb, 1, _BT)
    grid_spec = pltpu.PrefetchScalarGridSpec(
        num_scalar_prefetch=1,
        grid=(nb,),
        in_specs=[
            pl.BlockSpec((_BT, d), lambda i, be: (i, 0)),
            pl.BlockSpec((1, d, f), lambda i, be: (be[i], 0, 0)),
            pl.BlockSpec((1, 1, f), lambda i, be: (be[i], 0, 0)),
            pl.BlockSpec((1, f, d), lambda i, be: (be[i], 0, 0)),
            pl.BlockSpec((1, 1, d), lambda i, be: (be[i], 0, 0)),
            pl.BlockSpec((1, 1, _BT), lambda i, be: (i, 0, 0)),
        ],
        out_specs=pl.BlockSpec((_BT, d), lambda i, be: (i, 0)),
    )
    return pl.pallas_call(
        _ffn_body,
        grid_spec=grid_spec,
        out_shape=jax.ShapeDtypeStruct((npos, d), jnp.float32),
    )(block_expert, xs, W1, b1.reshape(e, 1, f), W2, b2.reshape(e, 1, d), ws3)


def kernel(x, Wg, W1, b1, W2, b2):
    b, s, d = x.shape
    t = b * s
    e = Wg.shape[1]
    p = t * 2
    nb = p // _BT + e
    npos = nb * _BT

    xf = jnp.reshape(x, (t, d))

    e0, e1, w0, w1 = _gating(xf, Wg)

    tok_pos, ws_pos, block_expert, gidx_a, gidx_b = _route(e0, e1, w0, w1, e)

    mesh = plsc.VectorSubcoreMesh(
        core_axis_name="c", subcore_axis_name="s",
        num_cores=_NC, num_subcores=_NS)
    per_w = npos // _NW

    @functools.partial(
        pl.kernel,
        mesh=mesh,
        out_type=jax.ShapeDtypeStruct((npos, d), jnp.float32),
        scratch_types=[
            pltpu.VMEM((_GC,), jnp.int32),
            pltpu.VMEM((_GC, d), jnp.float32),
            pltpu.SemaphoreType.DMA,
        ],
    )
    def dispatch(x_hbm, idx_hbm, xs_hbm, idx_v, rows_v, sem):
        wid = lax.axis_index("s") * _NC + lax.axis_index("c")
        base = wid * per_w
        for c in range(per_w // _GC):
            off = base + c * _GC
            pltpu.sync_copy(idx_hbm.at[pl.ds(off, _GC)], idx_v)
            pltpu.async_copy(x_hbm.at[idx_v], rows_v, sem).wait()
            pltpu.sync_copy(rows_v, xs_hbm.at[pl.ds(off, _GC)])

    xs = dispatch(xf, tok_pos)

    ys = _ffn(block_expert, xs, W1, b1, W2, b2, ws_pos)

    tpw = t // _NW

    @functools.partial(
        pl.kernel,
        mesh=mesh,
        out_type=jax.ShapeDtypeStruct((t, d), jnp.float32),
        scratch_types=[
            pltpu.VMEM((_CC,), jnp.int32),
            pltpu.VMEM((_CC,), jnp.int32),
            pltpu.VMEM((_CC, d), jnp.float32),
            pltpu.VMEM((_CC, d), jnp.float32),
            pltpu.VMEM((_CC, d), jnp.float32),
            pltpu.SemaphoreType.DMA,
            pltpu.SemaphoreType.DMA,
        ],
    )
    def combine(ys_hbm, ga_hbm, gb_hbm, out_hbm, ia_v, ib_v, ra_v, rb_v, ro_v,
                sema, semb):
        wid = lax.axis_index("s") * _NC + lax.axis_index("c")
        base = wid * tpw
        nvec = d // 16
        for c in range(tpw // _CC):
            off = base + c * _CC
            pltpu.sync_copy(ga_hbm.at[pl.ds(off, _CC)], ia_v)
            pltpu.sync_copy(gb_hbm.at[pl.ds(off, _CC)], ib_v)
            cpa = pltpu.async_copy(ys_hbm.at[ia_v], ra_v, sema)
            cpb = pltpu.async_copy(ys_hbm.at[ib_v], rb_v, semb)
            cpa.wait()
            cpb.wait()

            def body(i, _):
                r = i // nvec
                j = (i - r * nvec) * 16
                ro_v[r, pl.ds(j, 16)] = ra_v[r, pl.ds(j, 16)] + rb_v[r, pl.ds(j, 16)]
                return 0

            lax.fori_loop(0, _CC * nvec, body, 0)
            pltpu.sync_copy(ro_v, out_hbm.at[pl.ds(off, _CC)])

    out = combine(ys, gidx_a, gidx_b)
    return out.reshape(b, s, d)


# speedup vs baseline: 1.1948x; 1.0696x over previous
import functools

import jax
import jax.numpy as jnp
from jax import lax
from jax.experimental import pallas as pl
from jax.experimental.pallas import tpu as pltpu
from jax.experimental.pallas import tpu_sc as plsc

_NC, _NS = 2, 16
_NW = _NC * _NS

_BT = 256
_BTG = 512
_GC = 40
_CC = 16


def _gate_body(e_static, x_ref, wg_ref, e0_ref, e1_ref, w0_ref, w1_ref):
    logits = jnp.dot(x_ref[...], wg_ref[...], preferred_element_type=jnp.float32)
    col = lax.broadcasted_iota(jnp.int32, logits.shape, 1)
    neg = jnp.float32(-1e30)
    l0 = jnp.where(col < e_static, logits, neg)
    m1 = jnp.max(l0, axis=1, keepdims=True)
    i1 = jnp.min(jnp.where(l0 == m1, col, logits.shape[1]), axis=1, keepdims=True)
    lm = jnp.where(col == i1, neg, l0)
    m2 = jnp.max(lm, axis=1, keepdims=True)
    i2 = jnp.min(jnp.where(lm == m2, col, logits.shape[1]), axis=1, keepdims=True)
    t = jnp.exp(m2 - m1)
    w0 = 1.0 / (1.0 + t)
    e0_ref[0, 0, :] = i1[:, 0]
    e1_ref[0, 0, :] = i2[:, 0]
    w0_ref[0, 0, :] = w0[:, 0]
    w1_ref[0, 0, :] = 1.0 - w0[:, 0]


def _ffn_body(be_ref, xs_ref, w1_ref, b1_ref, w2_ref, b2_ref, ws_ref, ys_ref):
    xb = xs_ref[...]
    h = jnp.dot(xb, w1_ref[0], preferred_element_type=jnp.float32) + b1_ref[0]
    g = jax.nn.gelu(h)
    y = jnp.dot(g, w2_ref[0], preferred_element_type=jnp.float32) + b2_ref[0]
    ys_ref[...] = y * ws_ref[0, 0][:, None]


def _gating(xf, Wg):
    t, d = xf.shape
    e = Wg.shape[1]
    lanes = 128
    wg_pad = jnp.pad(Wg, ((0, 0), (0, lanes - e)))
    nbg = t // _BTG
    e0, e1, w0, w1 = pl.pallas_call(
        functools.partial(_gate_body, e),
        grid=(nbg,),
        in_specs=[
            pl.BlockSpec((_BTG, d), lambda i: (i, 0)),
            pl.BlockSpec((d, lanes), lambda i: (0, 0)),
        ],
        out_specs=[
            pl.BlockSpec((1, 1, _BTG), lambda i: (i, 0, 0)),
            pl.BlockSpec((1, 1, _BTG), lambda i: (i, 0, 0)),
            pl.BlockSpec((1, 1, _BTG), lambda i: (i, 0, 0)),
            pl.BlockSpec((1, 1, _BTG), lambda i: (i, 0, 0)),
        ],
        out_shape=[
            jax.ShapeDtypeStruct((nbg, 1, _BTG), jnp.int32),
            jax.ShapeDtypeStruct((nbg, 1, _BTG), jnp.int32),
            jax.ShapeDtypeStruct((nbg, 1, _BTG), jnp.float32),
            jax.ShapeDtypeStruct((nbg, 1, _BTG), jnp.float32),
        ],
    )(xf, wg_pad)
    return e0.reshape(t), e1.reshape(t), w0.reshape(t), w1.reshape(t)


def _route(e0, e1, w0, w1, e):
    t = e0.shape[0]
    p = t * 2
    nb = p // _BT + e
    npos = nb * _BT
    flat_e = jnp.stack([e0, e1], axis=1).reshape(p)
    flat_w = jnp.stack([w0, w1], axis=1).reshape(p)
    oh = (flat_e[:, None] == jnp.arange(e, dtype=jnp.int32)[None, :]).astype(jnp.int32)
    csum = jnp.cumsum(oh, axis=0)
    counts = csum[-1]
    rank = jnp.take_along_axis(csum, flat_e[:, None], axis=1)[:, 0] - 1
    nblk = (counts + _BT - 1) // _BT
    blk_start = jnp.concatenate(
        [jnp.zeros((1,), jnp.int32), jnp.cumsum(nblk)[:-1].astype(jnp.int32)])
    pos = blk_start[flat_e] * _BT + rank
    tok = jnp.arange(p, dtype=jnp.int32) // 2
    tok_pos = jnp.zeros((npos,), jnp.int32).at[pos].set(tok)
    ws_pos = jnp.zeros((npos,), jnp.float32).at[pos].set(flat_w)
    block_expert = jnp.minimum(
        (jnp.arange(nb, dtype=jnp.int32)[:, None] >= blk_start[None, :])
        .astype(jnp.int32).sum(axis=1) - 1,
        e - 1).astype(jnp.int32)
    return tok_pos, ws_pos, block_expert, pos


def _ffn(block_expert, xs, W1, b1, W2, b2, ws_pos):
    npos, d = xs.shape
    e, _, f = W1.shape
    nb = npos // _BT
    ws3 = ws_pos.reshape(nb, 1, _BT)
    grid_spec = pltpu.PrefetchScalarGridSpec(
        num_scalar_prefetch=1,
        grid=(nb,),
        in_specs=[
            pl.BlockSpec((_BT, d), lambda i, be: (i, 0)),
            pl.BlockSpec((1, d, f), lambda i, be: (be[i], 0, 0)),
            pl.BlockSpec((1, 1, f), lambda i, be: (be[i], 0, 0)),
            pl.BlockSpec((1, f, d), lambda i, be: (be[i], 0, 0)),
            pl.BlockSpec((1, 1, d), lambda i, be: (be[i], 0, 0)),
            pl.BlockSpec((1, 1, _BT), lambda i, be: (i, 0, 0)),
        ],
        out_specs=pl.BlockSpec((_BT, d), lambda i, be: (i, 0)),
    )
    return pl.pallas_call(
        _ffn_body,
        grid_spec=grid_spec,
        out_shape=jax.ShapeDtypeStruct((npos, d), jnp.float32),
    )(block_expert, xs, W1, b1.reshape(e, 1, f), W2, b2.reshape(e, 1, d), ws3)


def kernel(x, Wg, W1, b1, W2, b2):
    b, s, d = x.shape
    t = b * s
    e = Wg.shape[1]
    p = t * 2
    nb = p // _BT + e
    npos = nb * _BT

    xf = jnp.reshape(x, (t, d))

    e0, e1, w0, w1 = _gating(xf, Wg)

    tok_pos, ws_pos, block_expert, pos = _route(e0, e1, w0, w1, e)

    mesh = plsc.VectorSubcoreMesh(
        core_axis_name="c", subcore_axis_name="s",
        num_cores=_NC, num_subcores=_NS)
    per_w = npos // _NW
    nch = per_w // _GC

    @functools.partial(
        pl.kernel,
        mesh=mesh,
        out_type=jax.ShapeDtypeStruct((npos, d), jnp.float32),
        scratch_types=[
            pltpu.VMEM((per_w,), jnp.int32),
            pltpu.VMEM((_GC, d), jnp.float32),
            pltpu.VMEM((_GC, d), jnp.float32),
            pltpu.SemaphoreType.DMA,
            pltpu.SemaphoreType.DMA,
            pltpu.SemaphoreType.DMA,
            pltpu.SemaphoreType.DMA,
        ],
    )
    def dispatch(x_hbm, idx_hbm, xs_hbm, idx_v, buf0, buf1, g0, g1, w0s, w1s):
        wid = lax.axis_index("s") * _NC + lax.axis_index("c")
        base = wid * per_w
        pltpu.sync_copy(idx_hbm.at[pl.ds(base, per_w)], idx_v)
        bufs = (buf0, buf1)
        gsem = (g0, g1)
        wsem = (w0s, w1s)
        gcp = [None, None]
        wcp = [None, None]
        for c in range(nch):
            p = c & 1
            if wcp[p] is not None:
                wcp[p].wait()
            gcp[p] = pltpu.async_copy(
                x_hbm.at[idx_v.at[pl.ds(c * _GC, _GC)]], bufs[p], gsem[p])
            if c >= 1:
                q = 1 - p
                gcp[q].wait()
                wcp[q] = pltpu.async_copy(
                    bufs[q], xs_hbm.at[pl.ds(base + (c - 1) * _GC, _GC)], wsem[q])
        pl_ = (nch - 1) & 1
        gcp[pl_].wait()
        wcp[pl_] = pltpu.async_copy(
            bufs[pl_], xs_hbm.at[pl.ds(base + (nch - 1) * _GC, _GC)], wsem[pl_])
        for p in (0, 1):
            if wcp[p] is not None:
                wcp[p].wait()

    xs = dispatch(xf, tok_pos)

    ys = _ffn(block_expert, xs, W1, b1, W2, b2, ws_pos)

    tpw = t // _NW
    ncc = tpw // _CC
    nvec = d // 16

    @functools.partial(
        pl.kernel,
        mesh=mesh,
        out_type=jax.ShapeDtypeStruct((t, d), jnp.float32),
        scratch_types=[
            pltpu.VMEM((2 * tpw,), jnp.int32),
            pltpu.VMEM((2 * _CC, d), jnp.float32),
            pltpu.VMEM((2 * _CC, d), jnp.float32),
            pltpu.VMEM((_CC, d), jnp.float32),
            pltpu.VMEM((_CC, d), jnp.float32),
            pltpu.SemaphoreType.DMA,
            pltpu.SemaphoreType.DMA,
            pltpu.SemaphoreType.DMA,
            pltpu.SemaphoreType.DMA,
        ],
    )
    def combine(ys_hbm, pos_hbm, out_hbm, idx_v, ga0, ga1, o0, o1,
                sg0, sg1, so0, so1):
        wid = lax.axis_index("s") * _NC + lax.axis_index("c")
        base = wid * tpw
        pltpu.sync_copy(pos_hbm.at[pl.ds(2 * base, 2 * tpw)], idx_v)
        gbufs = (ga0, ga1)
        obufs = (o0, o1)
        gsem = (sg0, sg1)
        osem = (so0, so1)
        gcp = [None, None]
        ocp = [None, None]
        gcp[0] = pltpu.async_copy(
            ys_hbm.at[idx_v.at[pl.ds(0, 2 * _CC)]], gbufs[0], gsem[0])
        for c in range(ncc):
            p = c & 1
            q = 1 - p
            if c + 1 < ncc:
                gcp[q] = pltpu.async_copy(
                    ys_hbm.at[idx_v.at[pl.ds((c + 1) * 2 * _CC, 2 * _CC)]],
                    gbufs[q], gsem[q])
            gcp[p].wait()
            if ocp[p] is not None:
                ocp[p].wait()
            gb = gbufs[p]
            ob = obufs[p]
            for r in range(_CC):
                @plsc.parallel_loop(0, nvec, unroll=8)
                def _add(i, gb=gb, ob=ob, r=r):
                    sl = pl.ds(i * 16, 16)
                    ob[r, sl] = gb[2 * r, sl] + gb[2 * r + 1, sl]
            ocp[p] = pltpu.async_copy(
                ob, out_hbm.at[pl.ds(base + c * _CC, _CC)], osem[p])
        for p in (0, 1):
            if ocp[p] is not None:
                ocp[p].wait()

    out = combine(ys, pos)
    return out.reshape(b, s, d)
